# baseline (speedup 1.0000x reference)
import functools
import jax
import jax.numpy as jnp
from jax import lax
from jax.experimental import pallas as pl
from jax.experimental.pallas import tpu as pltpu
from jax.experimental.pallas import tpu_sc as plsc

VOCAB = 1000
B, T = 1024, 50
TP = 64
CA = 896
CT = VOCAB - CA
TA = 48

NC = 2
NS = 16
NW = NC * NS
NB = B // NW

_mesh = plsc.VectorSubcoreMesh(
    core_axis_name="c", subcore_axis_name="s", num_cores=NC, num_subcores=NS
)


@functools.partial(
    pl.kernel,
    out_type=jax.ShapeDtypeStruct((B, T, VOCAB), jnp.float32),
    mesh=_mesh,
    scratch_types=[
        pltpu.VMEM((NB * TP,), jnp.int32),
        pltpu.VMEM((TP, CA), jnp.float32),
        pltpu.VMEM((TP, 128), jnp.float32),
        pltpu.VMEM((T, CT), jnp.float32),
        pltpu.VMEM((T - TA, CA), jnp.float32),
        pltpu.SemaphoreType.DMA,
        pltpu.SemaphoreType.DMA,
    ],
)
def _sc_gather(
    tableA_hbm, tableT_hbm, idx_hbm, out_hbm,
    idx_v, bufA, bufT, bufT2, buf2, semA, semT,
):
    wid = lax.axis_index("s") * NC + lax.axis_index("c")
    b0 = wid * NB
    pltpu.sync_copy(idx_hbm.at[pl.ds(b0 * TP, NB * TP)], idx_v)

    @pl.loop(0, NB)
    def _batch(b):
        ids = idx_v.at[pl.ds(b * TP, TP)]
        cpA = pltpu.make_async_copy(tableA_hbm.at[ids], bufA, semA)
        cpT = pltpu.make_async_copy(tableT_hbm.at[ids], bufT, semT)
        cpA.start()
        cpA.wait()
        cpT.start()
        cpT.wait()

        for j in range(T - TA):
            for k in range(CA // 16):
                buf2[j, pl.ds(16 * k, 16)] = bufA[TA + j, pl.ds(16 * k, 16)]

        @pl.loop(0, T)
        def _row(r):
            for k in range(6):
                bufT2[r, pl.ds(16 * k, 16)] = bufT[r, pl.ds(16 * k, 16)]
            bufT2[r, pl.ds(CT - 16, 16)] = bufT[r, pl.ds(CT - 16, 16)]

        out_b = out_hbm.at[b0 + b]
        pltpu.sync_copy(bufA.at[pl.ds(0, TA), :], out_b.at[pl.ds(0, TA), pl.ds(0, CA)])
        pltpu.sync_copy(buf2, out_b.at[pl.ds(TA, T - TA), pl.ds(0, CA)])
        pltpu.sync_copy(bufT2, out_b.at[:, pl.ds(CA, CT)])


def kernel(idx, table):
    idx_p = jnp.pad(idx.astype(jnp.int32), ((0, 0), (0, TP - T))).reshape(-1)
    tableA = table[:, :CA]
    tableT = jnp.pad(table[:, CA:], ((0, 0), (0, 128 - CT)))
    return _sc_gather(tableA, tableT, idx_p)


# speedup vs baseline: 1.8120x; 1.8120x over previous
import functools
import jax
import jax.numpy as jnp
from jax import lax
from jax.experimental import pallas as pl
from jax.experimental.pallas import tpu as pltpu
from jax.experimental.pallas import tpu_sc as plsc

VOCAB = 1000
B, T = 1024, 50

NC = 2
NS = 16
NW = NC * NS
NB = B // NW

_mesh = plsc.VectorSubcoreMesh(
    core_axis_name="c", subcore_axis_name="s", num_cores=NC, num_subcores=NS
)


@functools.partial(
    pl.kernel,
    out_type=jax.ShapeDtypeStruct((B * T, VOCAB), jnp.float32),
    mesh=_mesh,
    scratch_types=[
        pltpu.VMEM((NB, T), jnp.int32),
        pltpu.VMEM((T, VOCAB), jnp.float32),
        pltpu.SemaphoreType.DMA,
    ],
    compiler_params=pltpu.CompilerParams(use_tc_tiling_on_sc=False),
)
def _sc_gather(table_hbm, idx_hbm, out_hbm, idx_v, buf, gsem):
    wid = lax.axis_index("s") * NC + lax.axis_index("c")
    b0 = wid * NB
    pltpu.sync_copy(idx_hbm.at[pl.ds(b0, NB)], idx_v)

    @pl.loop(0, NB)
    def _batch(b):
        pltpu.async_copy(table_hbm.at[idx_v.at[b]], buf, gsem).wait()
        pltpu.sync_copy(buf, out_hbm.at[pl.ds((b0 + b) * T, T)])


def kernel(idx, table):
    out = _sc_gather(table, idx.astype(jnp.int32))
    return out.reshape(B, T, VOCAB)
